# baseline (speedup 1.0000x reference)
import functools

import jax
import jax.numpy as jnp
from jax import lax
from jax.experimental import pallas as pl
from jax.experimental.pallas import tpu as pltpu
from jax.experimental.pallas import tpu_sc as plsc

_N = 32

_RGB_D = 384 * 384 * 3
_RGB_CHUNK = 2304
_RGB_NC = _RGB_D // _RGB_CHUNK
_RGB_GROUPS = _RGB_NC // 16

_MASK_D = 384 * 384
_MASK_CHUNK = 1024
_MASK_NC = _MASK_D // _MASK_CHUNK
_MASK_GROUPS = _MASK_NC // 16

_SMALL_W = 128


def _sc_gather(rgb2, mask2, small2, idx2, rgb_cidx, mask_cidx):
    info = plsc.get_sparse_core_info()
    nc = info.num_cores
    mesh = plsc.VectorSubcoreMesh(core_axis_name="c", subcore_axis_name="s")

    out_type = (
        jax.ShapeDtypeStruct((_N, _RGB_NC, _RGB_CHUNK), jnp.float32),
        jax.ShapeDtypeStruct((_N, _MASK_NC, _MASK_CHUNK), jnp.float32),
        jax.ShapeDtypeStruct((_N, _SMALL_W), jnp.float32),
    )
    scratch = [
        pltpu.VMEM((_N // 16, 16), jnp.int32),
        pltpu.VMEM((_RGB_GROUPS, 16), jnp.int32),
        pltpu.VMEM((_MASK_GROUPS, 16), jnp.int32),
        pltpu.VMEM((16, _RGB_CHUNK), jnp.float32),
        pltpu.VMEM((16, _MASK_CHUNK), jnp.float32),
        pltpu.VMEM((16, _SMALL_W), jnp.float32),
        pltpu.SemaphoreType.DMA,
    ]

    @functools.partial(
        pl.kernel, out_type=out_type, mesh=mesh, scratch_types=scratch
    )
    def gather_kernel(
        rgb_hbm, mask_hbm, small_hbm, idx_hbm, rgb_cidx_hbm, mask_cidx_hbm,
        rgb_out, mask_out, small_out,
        idx_v, cidx_r_v, cidx_m_v, rgb_buf, mask_buf, small_buf, sem,
    ):
        w = lax.axis_index("s") * nc + lax.axis_index("c")
        pltpu.sync_copy(idx_hbm, idx_v)
        pltpu.sync_copy(rgb_cidx_hbm.at[w], cidx_r_v)
        pltpu.sync_copy(mask_cidx_hbm.at[w], cidx_m_v)

        for g in range(_RGB_GROUPS):
            pltpu.async_copy(rgb_hbm.at[cidx_r_v.at[g]], rgb_buf, sem).wait()
            pltpu.sync_copy(rgb_buf, rgb_out.at[w, pl.ds(g * 16, 16)])

        for g in range(_MASK_GROUPS):
            pltpu.async_copy(mask_hbm.at[cidx_m_v.at[g]], mask_buf, sem).wait()
            pltpu.sync_copy(mask_buf, mask_out.at[w, pl.ds(g * 16, 16)])

        @pl.when(w < _N // 16)
        def _():
            pltpu.async_copy(small_hbm.at[idx_v.at[w]], small_buf, sem).wait()
            pltpu.sync_copy(small_buf, small_out.at[pl.ds(w * 16, 16)])

    return gather_kernel(rgb2, mask2, small2, idx2, rgb_cidx, mask_cidx)


def kernel(rgb_list, mask_list, K_list, pose_base_list, pose_rest_list,
           global_trans_list, index):
    f = rgb_list.shape[0]
    rgb2 = rgb_list.reshape(f * _RGB_NC, _RGB_CHUNK)
    mask2 = mask_list.reshape(f * _MASK_NC, _MASK_CHUNK)

    small2 = jnp.zeros((f, _SMALL_W), jnp.float32)
    small2 = small2.at[:, 0:9].set(K_list.reshape(f, 9))
    small2 = small2.at[:, 9:12].set(pose_base_list.reshape(f, 3))
    small2 = small2.at[:, 12:81].set(pose_rest_list.reshape(f, 69))
    small2 = small2.at[:, 81:84].set(global_trans_list)

    ar = jnp.arange(_RGB_NC, dtype=jnp.int32)
    rgb_cidx = (index[:, None] * _RGB_NC + ar).reshape(_N, _RGB_GROUPS, 16)
    am = jnp.arange(_MASK_NC, dtype=jnp.int32)
    mask_cidx = (index[:, None] * _MASK_NC + am).reshape(_N, _MASK_GROUPS, 16)
    idx2 = index.reshape(_N // 16, 16)

    rgb_o, mask_o, small_o = _sc_gather(
        rgb2, mask2, small2, idx2, rgb_cidx, mask_cidx
    )
    return (
        rgb_o.reshape(_N, 384, 384, 3),
        mask_o.reshape(_N, 384, 384),
        small_o[:, 0:9].reshape(_N, 3, 3),
        small_o[:, 9:12].reshape(_N, 1, 3),
        small_o[:, 12:81].reshape(_N, 23, 3),
        small_o[:, 81:84],
        index,
    )


# speedup vs baseline: 67.2731x; 67.2731x over previous
import functools

import jax
import jax.numpy as jnp
from jax import lax
from jax.experimental import pallas as pl
from jax.experimental.pallas import tpu as pltpu
from jax.experimental.pallas import tpu_sc as plsc

_N = 32
_F = 100
_RPF = 384
_RGB_W = 384 * 3
_MASK_W = 384
_GROUP = 32
_NG = _RPF // _GROUP
_SMALL_W = 128


def _sc_gather(rgb2, mask2, small2, idx2, cidx):
    info = plsc.get_sparse_core_info()
    nc = info.num_cores
    mesh = plsc.VectorSubcoreMesh(core_axis_name="c", subcore_axis_name="s")

    out_type = (
        jax.ShapeDtypeStruct((_N * _RPF, _RGB_W), jnp.float32),
        jax.ShapeDtypeStruct((_N * _RPF, _MASK_W), jnp.float32),
        jax.ShapeDtypeStruct((_N, _SMALL_W), jnp.float32),
    )
    scratch = [
        pltpu.VMEM((_N // 16, 16), jnp.int32),
        pltpu.VMEM((_NG, _GROUP), jnp.int32),
        pltpu.VMEM((_GROUP, _RGB_W), jnp.float32),
        pltpu.VMEM((_GROUP, _MASK_W), jnp.float32),
        pltpu.VMEM((16, _SMALL_W), jnp.float32),
        pltpu.SemaphoreType.DMA,
    ]

    @functools.partial(
        pl.kernel, out_type=out_type, mesh=mesh, scratch_types=scratch
    )
    def gather_kernel(
        rgb_hbm, mask_hbm, small_hbm, idx_hbm, cidx_hbm,
        rgb_out, mask_out, small_out,
        idx_v, cidx_v, rgb_buf, mask_buf, small_buf, sem,
    ):
        w = lax.axis_index("s") * nc + lax.axis_index("c")
        pltpu.sync_copy(idx_hbm, idx_v)
        pltpu.sync_copy(cidx_hbm.at[w], cidx_v)

        for g in range(_NG):
            rows = cidx_v.at[g]
            base = w * _RPF + g * _GROUP
            pltpu.async_copy(rgb_hbm.at[rows], rgb_buf, sem).wait()
            pltpu.sync_copy(rgb_buf, rgb_out.at[pl.ds(base, _GROUP)])
            pltpu.async_copy(mask_hbm.at[rows], mask_buf, sem).wait()
            pltpu.sync_copy(mask_buf, mask_out.at[pl.ds(base, _GROUP)])

        @pl.when(w < _N // 16)
        def _():
            pltpu.async_copy(small_hbm.at[idx_v.at[w]], small_buf, sem).wait()
            pltpu.sync_copy(small_buf, small_out.at[pl.ds(w * 16, 16)])

    return gather_kernel(rgb2, mask2, small2, idx2, cidx)


def kernel(rgb_list, mask_list, K_list, pose_base_list, pose_rest_list,
           global_trans_list, index):
    f = rgb_list.shape[0]
    rgb2 = rgb_list.reshape(f * _RPF, _RGB_W)
    mask2 = mask_list.reshape(f * _RPF, _MASK_W)

    small2 = jnp.zeros((f, _SMALL_W), jnp.float32)
    small2 = small2.at[:, 0:9].set(K_list.reshape(f, 9))
    small2 = small2.at[:, 9:12].set(pose_base_list.reshape(f, 3))
    small2 = small2.at[:, 12:81].set(pose_rest_list.reshape(f, 69))
    small2 = small2.at[:, 81:84].set(global_trans_list)

    ar = jnp.arange(_RPF, dtype=jnp.int32)
    cidx = (index[:, None] * _RPF + ar).reshape(_N, _NG, _GROUP)
    idx2 = index.reshape(_N // 16, 16)

    rgb_o, mask_o, small_o = _sc_gather(rgb2, mask2, small2, idx2, cidx)
    return (
        rgb_o.reshape(_N, 384, 384, 3),
        mask_o.reshape(_N, 384, 384),
        small_o[:, 0:9].reshape(_N, 3, 3),
        small_o[:, 9:12].reshape(_N, 1, 3),
        small_o[:, 12:81].reshape(_N, 23, 3),
        small_o[:, 81:84],
        index,
    )


# speedup vs baseline: 452.2235x; 6.7222x over previous
import functools

import jax
import jax.numpy as jnp
from jax import lax
from jax.experimental import pallas as pl
from jax.experimental.pallas import tpu as pltpu
from jax.experimental.pallas import tpu_sc as plsc

_N = 32
_F = 100
_W = 384

_RGB_RPF = 3 * 384
_RGB_G = 128
_RGB_NG = _RGB_RPF // _RGB_G

_MASK_RPF = 384
_MASK_G = 64
_MASK_NG = _MASK_RPF // _MASK_G

_SMALL_W = 128


def _sc_gather(rgb2, mask2, small2, idx2, rgb_cidx, mask_cidx):
    info = plsc.get_sparse_core_info()
    nc = info.num_cores
    mesh = plsc.VectorSubcoreMesh(core_axis_name="c", subcore_axis_name="s")

    out_type = (
        jax.ShapeDtypeStruct((_N * _RGB_RPF, _W), jnp.float32),
        jax.ShapeDtypeStruct((_N * _MASK_RPF, _W), jnp.float32),
        jax.ShapeDtypeStruct((_N, _SMALL_W), jnp.float32),
    )
    scratch = [
        pltpu.VMEM((_N // 16, 16), jnp.int32),
        pltpu.VMEM((_RGB_NG, _RGB_G), jnp.int32),
        pltpu.VMEM((_MASK_NG, _MASK_G), jnp.int32),
        pltpu.VMEM((_RGB_G, _W), jnp.float32),
        pltpu.VMEM((_MASK_G, _W), jnp.float32),
        pltpu.VMEM((16, _SMALL_W), jnp.float32),
        pltpu.SemaphoreType.DMA,
    ]

    @functools.partial(
        pl.kernel, out_type=out_type, mesh=mesh, scratch_types=scratch
    )
    def gather_kernel(
        rgb_hbm, mask_hbm, small_hbm, idx_hbm, rgb_cidx_hbm, mask_cidx_hbm,
        rgb_out, mask_out, small_out,
        idx_v, cidx_r_v, cidx_m_v, rgb_buf, mask_buf, small_buf, sem,
    ):
        w = lax.axis_index("s") * nc + lax.axis_index("c")
        pltpu.sync_copy(idx_hbm, idx_v)
        pltpu.sync_copy(rgb_cidx_hbm.at[w], cidx_r_v)
        pltpu.sync_copy(mask_cidx_hbm.at[w], cidx_m_v)

        for g in range(_RGB_NG):
            pltpu.async_copy(rgb_hbm.at[cidx_r_v.at[g]], rgb_buf, sem).wait()
            pltpu.sync_copy(
                rgb_buf, rgb_out.at[pl.ds(w * _RGB_RPF + g * _RGB_G, _RGB_G)]
            )

        for g in range(_MASK_NG):
            pltpu.async_copy(mask_hbm.at[cidx_m_v.at[g]], mask_buf, sem).wait()
            pltpu.sync_copy(
                mask_buf, mask_out.at[pl.ds(w * _MASK_RPF + g * _MASK_G, _MASK_G)]
            )

        @pl.when(w < _N // 16)
        def _():
            pltpu.async_copy(small_hbm.at[idx_v.at[w]], small_buf, sem).wait()
            pltpu.sync_copy(small_buf, small_out.at[pl.ds(w * 16, 16)])

    return gather_kernel(rgb2, mask2, small2, idx2, rgb_cidx, mask_cidx)


def kernel(rgb_list, mask_list, K_list, pose_base_list, pose_rest_list,
           global_trans_list, index):
    f = rgb_list.shape[0]
    rgb2 = rgb_list.transpose(0, 3, 1, 2).reshape(f * _RGB_RPF, _W)
    mask2 = mask_list.reshape(f * _MASK_RPF, _W)

    small2 = jnp.zeros((f, _SMALL_W), jnp.float32)
    small2 = small2.at[:, 0:9].set(K_list.reshape(f, 9))
    small2 = small2.at[:, 9:12].set(pose_base_list.reshape(f, 3))
    small2 = small2.at[:, 12:81].set(pose_rest_list.reshape(f, 69))
    small2 = small2.at[:, 81:84].set(global_trans_list)

    ar = jnp.arange(_RGB_RPF, dtype=jnp.int32)
    rgb_cidx = (index[:, None] * _RGB_RPF + ar).reshape(_N, _RGB_NG, _RGB_G)
    am = jnp.arange(_MASK_RPF, dtype=jnp.int32)
    mask_cidx = (index[:, None] * _MASK_RPF + am).reshape(_N, _MASK_NG, _MASK_G)
    idx2 = index.reshape(_N // 16, 16)

    rgb_o, mask_o, small_o = _sc_gather(
        rgb2, mask2, small2, idx2, rgb_cidx, mask_cidx
    )
    return (
        rgb_o.reshape(_N, 3, 384, 384).transpose(0, 2, 3, 1),
        mask_o.reshape(_N, 384, 384),
        small_o[:, 0:9].reshape(_N, 3, 3),
        small_o[:, 9:12].reshape(_N, 1, 3),
        small_o[:, 12:81].reshape(_N, 23, 3),
        small_o[:, 81:84],
        index,
    )


# speedup vs baseline: 505.4776x; 1.1178x over previous
import functools

import jax
import jax.numpy as jnp
from jax import lax
from jax.experimental import pallas as pl
from jax.experimental.pallas import tpu as pltpu
from jax.experimental.pallas import tpu_sc as plsc

_N = 32
_F = 100
_W = 384

_RGB_RPF = 3 * 384
_G = 128
_RGB_NG = _RGB_RPF // _G
_MASK_RPF = 384
_MASK_NG = _MASK_RPF // _G
_NG = _RGB_NG + _MASK_NG

_SMALL_W = 128


def _sc_gather(rgb2, mask2, small2, idx2, rgb_cidx, mask_cidx):
    info = plsc.get_sparse_core_info()
    nc = info.num_cores
    mesh = plsc.VectorSubcoreMesh(core_axis_name="c", subcore_axis_name="s")

    out_type = (
        jax.ShapeDtypeStruct((_N * _RGB_RPF, _W), jnp.float32),
        jax.ShapeDtypeStruct((_N * _MASK_RPF, _W), jnp.float32),
        jax.ShapeDtypeStruct((_N, _SMALL_W), jnp.float32),
    )
    scratch = [
        pltpu.VMEM((_N // 16, 16), jnp.int32),
        pltpu.VMEM((_RGB_NG, _G), jnp.int32),
        pltpu.VMEM((_MASK_NG, _G), jnp.int32),
        pltpu.VMEM((_G, _W), jnp.float32),
        pltpu.VMEM((_G, _W), jnp.float32),
        pltpu.VMEM((16, _SMALL_W), jnp.float32),
        pltpu.SemaphoreType.DMA,
        pltpu.SemaphoreType.DMA,
    ]

    @functools.partial(
        pl.kernel, out_type=out_type, mesh=mesh, scratch_types=scratch
    )
    def gather_kernel(
        rgb_hbm, mask_hbm, small_hbm, idx_hbm, rgb_cidx_hbm, mask_cidx_hbm,
        rgb_out, mask_out, small_out,
        idx_v, cidx_r_v, cidx_m_v, buf0, buf1, small_buf, rsem, wsem,
    ):
        w = lax.axis_index("s") * nc + lax.axis_index("c")
        pltpu.sync_copy(idx_hbm, idx_v)
        pltpu.sync_copy(rgb_cidx_hbm.at[w], cidx_r_v)
        pltpu.sync_copy(mask_cidx_hbm.at[w], cidx_m_v)
        bufs = (buf0, buf1)

        def src_of(i):
            if i < _RGB_NG:
                return rgb_hbm.at[cidx_r_v.at[i]]
            return mask_hbm.at[cidx_m_v.at[i - _RGB_NG]]

        def dst_of(i):
            if i < _RGB_NG:
                return rgb_out.at[pl.ds(w * _RGB_RPF + i * _G, _G)]
            return mask_out.at[pl.ds(w * _MASK_RPF + (i - _RGB_NG) * _G, _G)]

        gath = [None] * _NG
        wr = [None] * _NG
        gath[0] = pltpu.async_copy(src_of(0), bufs[0], rsem)
        for i in range(_NG):
            gath[i].wait()
            if i + 1 < _NG:
                if i >= 1:
                    wr[i - 1].wait()
                gath[i + 1] = pltpu.async_copy(
                    src_of(i + 1), bufs[(i + 1) % 2], rsem
                )
            wr[i] = pltpu.async_copy(bufs[i % 2], dst_of(i), wsem)

        @pl.when(w < _N // 16)
        def _():
            pltpu.async_copy(small_hbm.at[idx_v.at[w]], small_buf, rsem).wait()
            pltpu.sync_copy(small_buf, small_out.at[pl.ds(w * 16, 16)])

        wr[_NG - 2].wait()
        wr[_NG - 1].wait()

    return gather_kernel(rgb2, mask2, small2, idx2, rgb_cidx, mask_cidx)


def kernel(rgb_list, mask_list, K_list, pose_base_list, pose_rest_list,
           global_trans_list, index):
    f = rgb_list.shape[0]
    rgb2 = rgb_list.transpose(0, 3, 1, 2).reshape(f * _RGB_RPF, _W)
    mask2 = mask_list.reshape(f * _MASK_RPF, _W)

    small2 = jnp.zeros((f, _SMALL_W), jnp.float32)
    small2 = small2.at[:, 0:9].set(K_list.reshape(f, 9))
    small2 = small2.at[:, 9:12].set(pose_base_list.reshape(f, 3))
    small2 = small2.at[:, 12:81].set(pose_rest_list.reshape(f, 69))
    small2 = small2.at[:, 81:84].set(global_trans_list)

    ar = jnp.arange(_RGB_RPF, dtype=jnp.int32)
    rgb_cidx = (index[:, None] * _RGB_RPF + ar).reshape(_N, _RGB_NG, _G)
    am = jnp.arange(_MASK_RPF, dtype=jnp.int32)
    mask_cidx = (index[:, None] * _MASK_RPF + am).reshape(_N, _MASK_NG, _G)
    idx2 = index.reshape(_N // 16, 16)

    rgb_o, mask_o, small_o = _sc_gather(
        rgb2, mask2, small2, idx2, rgb_cidx, mask_cidx
    )
    return (
        rgb_o.reshape(_N, 3, 384, 384).transpose(0, 2, 3, 1),
        mask_o.reshape(_N, 384, 384),
        small_o[:, 0:9].reshape(_N, 3, 3),
        small_o[:, 9:12].reshape(_N, 1, 3),
        small_o[:, 12:81].reshape(_N, 23, 3),
        small_o[:, 81:84],
        index,
    )


# speedup vs baseline: 529.7474x; 1.0480x over previous
import functools

import jax
import jax.numpy as jnp
from jax import lax
from jax.experimental import pallas as pl
from jax.experimental.pallas import tpu as pltpu
from jax.experimental.pallas import tpu_sc as plsc

_N = 32
_F = 100
_W = 384

_RGB_RPF = 3 * 384
_G = 64
_NG = _RGB_RPF // _G
_NBUF = 4

_SMALL_W = 128


def _sc_gather(rgb2, small2, idx2, rgb_cidx):
    info = plsc.get_sparse_core_info()
    nc = info.num_cores
    mesh = plsc.VectorSubcoreMesh(core_axis_name="c", subcore_axis_name="s")

    out_type = (
        jax.ShapeDtypeStruct((_N * _RGB_RPF, _W), jnp.float32),
        jax.ShapeDtypeStruct((_N, _SMALL_W), jnp.float32),
    )
    scratch = [
        pltpu.VMEM((_N // 16, 16), jnp.int32),
        pltpu.VMEM((_NG, _G), jnp.int32),
        [pltpu.VMEM((_G, _W), jnp.float32) for _ in range(_NBUF)],
        pltpu.VMEM((16, _SMALL_W), jnp.float32),
        pltpu.SemaphoreType.DMA,
        pltpu.SemaphoreType.DMA,
    ]

    @functools.partial(
        pl.kernel, out_type=out_type, mesh=mesh, scratch_types=scratch
    )
    def gather_kernel(
        rgb_hbm, small_hbm, idx_hbm, rgb_cidx_hbm,
        rgb_out, small_out,
        idx_v, cidx_v, bufs, small_buf, rsem, wsem,
    ):
        w = lax.axis_index("s") * nc + lax.axis_index("c")
        pltpu.sync_copy(idx_hbm, idx_v)
        pltpu.sync_copy(rgb_cidx_hbm.at[w], cidx_v)

        def src_of(i):
            return rgb_hbm.at[cidx_v.at[i]]

        def dst_of(i):
            return rgb_out.at[pl.ds(w * _RGB_RPF + i * _G, _G)]

        gath = [None] * _NG
        wr = [None] * _NG
        gath[0] = pltpu.async_copy(src_of(0), bufs[0], rsem)
        gath[1] = pltpu.async_copy(src_of(1), bufs[1], rsem)
        for i in range(_NG):
            gath[i].wait()
            if i + 2 < _NG:
                if i >= 2:
                    wr[i - 2].wait()
                gath[i + 2] = pltpu.async_copy(
                    src_of(i + 2), bufs[(i + 2) % _NBUF], rsem
                )
            wr[i] = pltpu.async_copy(bufs[i % _NBUF], dst_of(i), wsem)

        @pl.when(w < _N // 16)
        def _():
            pltpu.async_copy(small_hbm.at[idx_v.at[w]], small_buf, rsem).wait()
            pltpu.sync_copy(small_buf, small_out.at[pl.ds(w * 16, 16)])

        for i in range(max(0, _NG - 4), _NG):
            wr[i].wait()

    return gather_kernel(rgb2, small2, idx2, rgb_cidx)


def _tc_mask_gather(mask_list, index):
    grid_spec = pltpu.PrefetchScalarGridSpec(
        num_scalar_prefetch=1,
        grid=(_N,),
        in_specs=[
            pl.BlockSpec((1, 384, 384), lambda i, idx_ref: (idx_ref[i], 0, 0))
        ],
        out_specs=pl.BlockSpec((1, 384, 384), lambda i, idx_ref: (i, 0, 0)),
    )

    def body(idx_ref, src_ref, dst_ref):
        dst_ref[...] = src_ref[...]

    return pl.pallas_call(
        body,
        grid_spec=grid_spec,
        out_shape=jax.ShapeDtypeStruct((_N, 384, 384), jnp.float32),
    )(index, mask_list)


def kernel(rgb_list, mask_list, K_list, pose_base_list, pose_rest_list,
           global_trans_list, index):
    f = rgb_list.shape[0]
    rgb2 = rgb_list.transpose(0, 3, 1, 2).reshape(f * _RGB_RPF, _W)

    small2 = jnp.zeros((f, _SMALL_W), jnp.float32)
    small2 = small2.at[:, 0:9].set(K_list.reshape(f, 9))
    small2 = small2.at[:, 9:12].set(pose_base_list.reshape(f, 3))
    small2 = small2.at[:, 12:81].set(pose_rest_list.reshape(f, 69))
    small2 = small2.at[:, 81:84].set(global_trans_list)

    ar = jnp.arange(_RGB_RPF, dtype=jnp.int32)
    rgb_cidx = (index[:, None] * _RGB_RPF + ar).reshape(_N, _NG, _G)
    idx2 = index.reshape(_N // 16, 16)

    rgb_o, small_o = _sc_gather(rgb2, small2, idx2, rgb_cidx)
    gt_mask = _tc_mask_gather(mask_list, index)
    return (
        rgb_o.reshape(_N, 3, 384, 384).transpose(0, 2, 3, 1),
        gt_mask,
        small_o[:, 0:9].reshape(_N, 3, 3),
        small_o[:, 9:12].reshape(_N, 1, 3),
        small_o[:, 12:81].reshape(_N, 23, 3),
        small_o[:, 81:84],
        index,
    )
